# baseline (speedup 1.0000x reference)
import jax
import jax.numpy as jnp
from jax import lax
from jax.experimental import pallas as pl
from jax.experimental.pallas import tpu as pltpu
from jax.experimental.pallas import tpu_sc as plsc

VOCAB = 49408
HIDDEN = 768
SEQ = 77
BATCH = 1024
LANES = 16
NVEC = HIDDEN // LANES

NUM_CORES = 2
NUM_SUBCORES = 16
NUM_WORKERS = NUM_CORES * NUM_SUBCORES
NUM_PAIRS = NUM_WORKERS // 2
BATCHES_PER_PAIR = BATCH // NUM_PAIRS

FULL_ROWS = 72
CHUNK_A = 40
CHUNK_B = 32
TAIL = 8
BATCH_BLOCK = 8


def _embed_body(ids_hbm, tids_hbm, tok_hbm, pos_hbm, out_hbm, tail_hbm,
                pos_v, idx_v, tidx_v, buf, tbuf, gsem, ssem, tgsem, tssem):
    cid = lax.axis_index("c")
    sid = lax.axis_index("s")
    wid = sid * NUM_CORES + cid
    pair = wid // 2
    role = wid % 2
    base_b = pair * BATCHES_PER_PAIR
    nb = BATCHES_PER_PAIR

    is_a = role == 0
    nrows = jnp.where(is_a, CHUNK_A, CHUNK_B)

    @pl.when(is_a)
    def _load_pos_a():
        pltpu.sync_copy(pos_hbm.at[pl.ds(0, CHUNK_A)], pos_v)

    @pl.when(~is_a)
    def _load_pos_b():
        pltpu.sync_copy(pos_hbm.at[pl.ds(CHUNK_A, CHUNK_A)], pos_v)

    def load_idx(b, slot):
        gb = base_b + b
        pltpu.sync_copy(ids_hbm.at[gb], idx_v.at[slot])
        @pl.when(~is_a)
        def _():
            pltpu.sync_copy(tids_hbm.at[gb], tidx_v.at[slot])

    def issue_gathers(b, slot):
        @pl.when(is_a)
        def _():
            pltpu.async_copy(
                tok_hbm.at[idx_v.at[slot].at[pl.ds(0, CHUNK_A)]],
                buf.at[slot], gsem)
        @pl.when(~is_a)
        def _():
            pltpu.async_copy(
                tok_hbm.at[idx_v.at[slot].at[pl.ds(CHUNK_A, CHUNK_B)]],
                buf.at[slot].at[pl.ds(0, CHUNK_B)], gsem)
            pltpu.async_copy(tok_hbm.at[tidx_v.at[slot]], tbuf.at[slot], tgsem)

    def wait_gathers(slot):
        @pl.when(is_a)
        def _():
            pltpu.make_async_copy(
                tok_hbm.at[idx_v.at[slot].at[pl.ds(0, CHUNK_A)]],
                buf.at[slot], gsem).wait()
        @pl.when(~is_a)
        def _():
            pltpu.make_async_copy(
                tok_hbm.at[idx_v.at[slot].at[pl.ds(CHUNK_A, CHUNK_B)]],
                buf.at[slot].at[pl.ds(0, CHUNK_B)], gsem).wait()
            pltpu.make_async_copy(
                tok_hbm.at[tidx_v.at[slot]], tbuf.at[slot], tgsem).wait()

    def issue_stores(b, slot):
        gb = base_b + b
        @pl.when(is_a)
        def _():
            pltpu.async_copy(buf.at[slot], out_hbm.at[gb, pl.ds(0, CHUNK_A)],
                             ssem)
        @pl.when(~is_a)
        def _():
            pltpu.async_copy(buf.at[slot].at[pl.ds(0, CHUNK_B)],
                             out_hbm.at[gb, pl.ds(CHUNK_A, CHUNK_B)], ssem)
            pltpu.async_copy(tbuf.at[slot], tail_hbm.at[gb], tssem)

    def drain_stores(slot):
        gb = base_b
        @pl.when(is_a)
        def _():
            pltpu.make_async_copy(buf.at[slot],
                                  out_hbm.at[gb, pl.ds(0, CHUNK_A)],
                                  ssem).wait()
        @pl.when(~is_a)
        def _():
            pltpu.make_async_copy(buf.at[slot].at[pl.ds(0, CHUNK_B)],
                                  out_hbm.at[gb, pl.ds(CHUNK_A, CHUNK_B)],
                                  ssem).wait()
            pltpu.make_async_copy(tbuf.at[slot], tail_hbm.at[gb], tssem).wait()

    def add_pos(slot):
        def row_body(r, carry):
            for c in range(NVEC):
                x = pos_v[r, pl.ds(c * LANES, LANES)]
                plsc.addupdate(buf.at[slot, r, pl.ds(c * LANES, LANES)], x)
            return carry
        lax.fori_loop(0, nrows, row_body, 0)

    load_idx(0, 0)
    issue_gathers(0, 0)

    def batch_body(b, carry):
        slot = lax.rem(b, 2)
        nslot = 1 - slot

        @pl.when(b + 1 < nb)
        def _prefetch():
            load_idx(b + 1, nslot)
            @pl.when(b >= 1)
            def _():
                drain_stores(nslot)
            issue_gathers(b + 1, nslot)

        wait_gathers(slot)
        add_pos(slot)
        issue_stores(b, slot)
        return carry

    lax.fori_loop(0, nb, batch_body, 0)
    drain_stores(lax.rem(nb - 2, 2))
    drain_stores(lax.rem(nb - 1, 2))


def _tail_body(x_ref, tail_ref, pos_ref, o_ref):
    o_ref[...] = tail_ref[...] + pos_ref[...][None, :, :]


def _tail_fix(out_sc, tail_tok, position_table):
    return pl.pallas_call(
        _tail_body,
        out_shape=jax.ShapeDtypeStruct((BATCH, SEQ, HIDDEN), jnp.float32),
        grid=(BATCH // BATCH_BLOCK,),
        in_specs=[
            pl.BlockSpec((1, TAIL, HIDDEN), lambda b: (b, 9, 0)),
            pl.BlockSpec((BATCH_BLOCK, TAIL, HIDDEN), lambda b: (b, 0, 0)),
            pl.BlockSpec((TAIL, HIDDEN), lambda b: (9, 0)),
        ],
        out_specs=pl.BlockSpec((BATCH_BLOCK, TAIL, HIDDEN), lambda b: (b, 9, 0)),
        input_output_aliases={0: 0},
    )(out_sc, tail_tok, position_table)


@jax.jit
def _embed(ids, token_table, position_table):
    tail_ids = jnp.pad(ids[:, FULL_ROWS:], ((0, 0), (0, TAIL - (SEQ - FULL_ROWS))))
    pos_pad = jnp.pad(position_table, ((0, TAIL * 10 - SEQ), (0, 0)))
    mesh = plsc.VectorSubcoreMesh(
        core_axis_name="c", subcore_axis_name="s",
        num_cores=NUM_CORES, num_subcores=NUM_SUBCORES,
    )
    f = pl.kernel(
        _embed_body,
        out_type=(
            jax.ShapeDtypeStruct((BATCH, SEQ, HIDDEN), jnp.float32),
            jax.ShapeDtypeStruct((BATCH, TAIL, HIDDEN), jnp.float32),
        ),
        mesh=mesh,
        scratch_types=[
            pltpu.VMEM((CHUNK_A, HIDDEN), jnp.float32),
            pltpu.VMEM((2, SEQ), jnp.int32),
            pltpu.VMEM((2, TAIL), jnp.int32),
            pltpu.VMEM((2, CHUNK_A, HIDDEN), jnp.float32),
            pltpu.VMEM((2, TAIL, HIDDEN), jnp.float32),
            pltpu.SemaphoreType.DMA,
            pltpu.SemaphoreType.DMA,
            pltpu.SemaphoreType.DMA,
            pltpu.SemaphoreType.DMA,
        ],
    )
    out_sc, tail_tok = f(ids, tail_ids, token_table, pos_pad)
    return _tail_fix(out_sc, tail_tok, position_table)


def kernel(input_ids, token_table, position_table):
    ids = input_ids.astype(jnp.int32)
    return _embed(ids, token_table, position_table)


# speedup vs baseline: 1.0017x; 1.0017x over previous
import jax
import jax.numpy as jnp
from jax import lax
from jax.experimental import pallas as pl
from jax.experimental.pallas import tpu as pltpu
from jax.experimental.pallas import tpu_sc as plsc

VOCAB = 49408
HIDDEN = 768
SEQ = 77
BATCH = 1024
LANES = 16
NVEC = HIDDEN // LANES

NUM_CORES = 2
NUM_SUBCORES = 16
NUM_WORKERS = NUM_CORES * NUM_SUBCORES
NUM_PAIRS = NUM_WORKERS // 2
BATCHES_PER_PAIR = BATCH // NUM_PAIRS

FULL_ROWS = 72
CHUNK_A = 40
CHUNK_B = 32
TAIL = 8
BATCH_BLOCK = 8


def _embed_body(ids_hbm, tids_hbm, tok_hbm, pos_hbm, out_hbm, tail_hbm,
                pos_v, idx_all, tidx_all, buf, tbuf, gsem, ssem, tgsem, tssem):
    cid = lax.axis_index("c")
    sid = lax.axis_index("s")
    wid = sid * NUM_CORES + cid
    pair = wid // 2
    role = wid % 2
    base_b = pair * BATCHES_PER_PAIR
    nb = BATCHES_PER_PAIR

    is_a = role == 0
    nrows = jnp.where(is_a, CHUNK_A, CHUNK_B)

    @pl.when(is_a)
    def _load_pos_a():
        pltpu.sync_copy(pos_hbm.at[pl.ds(0, CHUNK_A)], pos_v)

    @pl.when(~is_a)
    def _load_pos_b():
        pltpu.sync_copy(pos_hbm.at[pl.ds(CHUNK_A, CHUNK_A)], pos_v)

    pltpu.sync_copy(ids_hbm.at[pl.ds(base_b, nb)], idx_all)
    @pl.when(~is_a)
    def _load_tids():
        pltpu.sync_copy(tids_hbm.at[pl.ds(base_b, nb)], tidx_all)

    def issue_gathers(b, slot):
        @pl.when(is_a)
        def _():
            pltpu.async_copy(
                tok_hbm.at[idx_all.at[b].at[pl.ds(0, CHUNK_A)]],
                buf.at[slot], gsem)
        @pl.when(~is_a)
        def _():
            pltpu.async_copy(
                tok_hbm.at[idx_all.at[b].at[pl.ds(CHUNK_A, CHUNK_B)]],
                buf.at[slot].at[pl.ds(0, CHUNK_B)], gsem)
            pltpu.async_copy(tok_hbm.at[tidx_all.at[b]], tbuf.at[slot], tgsem)

    def wait_gathers(b, slot):
        @pl.when(is_a)
        def _():
            pltpu.make_async_copy(
                tok_hbm.at[idx_all.at[b].at[pl.ds(0, CHUNK_A)]],
                buf.at[slot], gsem).wait()
        @pl.when(~is_a)
        def _():
            pltpu.make_async_copy(
                tok_hbm.at[idx_all.at[b].at[pl.ds(CHUNK_A, CHUNK_B)]],
                buf.at[slot].at[pl.ds(0, CHUNK_B)], gsem).wait()
            pltpu.make_async_copy(
                tok_hbm.at[tidx_all.at[b]], tbuf.at[slot], tgsem).wait()

    def issue_stores(b, slot):
        gb = base_b + b
        @pl.when(is_a)
        def _():
            pltpu.async_copy(buf.at[slot], out_hbm.at[gb, pl.ds(0, CHUNK_A)],
                             ssem)
        @pl.when(~is_a)
        def _():
            pltpu.async_copy(buf.at[slot].at[pl.ds(0, CHUNK_B)],
                             out_hbm.at[gb, pl.ds(CHUNK_A, CHUNK_B)], ssem)
            pltpu.async_copy(tbuf.at[slot], tail_hbm.at[gb], tssem)

    def drain_stores(slot):
        gb = base_b
        @pl.when(is_a)
        def _():
            pltpu.make_async_copy(buf.at[slot],
                                  out_hbm.at[gb, pl.ds(0, CHUNK_A)],
                                  ssem).wait()
        @pl.when(~is_a)
        def _():
            pltpu.make_async_copy(buf.at[slot].at[pl.ds(0, CHUNK_B)],
                                  out_hbm.at[gb, pl.ds(CHUNK_A, CHUNK_B)],
                                  ssem).wait()
            pltpu.make_async_copy(tbuf.at[slot], tail_hbm.at[gb], tssem).wait()

    def add_pos(slot):
        def row_body(r, carry):
            for c in range(NVEC):
                x = pos_v[r, pl.ds(c * LANES, LANES)]
                plsc.addupdate(buf.at[slot, r, pl.ds(c * LANES, LANES)], x)
            return carry
        lax.fori_loop(0, nrows, row_body, 0)

    issue_gathers(0, 0)

    def batch_body(b, carry):
        slot = lax.rem(b, 2)
        nslot = 1 - slot

        @pl.when(b + 1 < nb)
        def _prefetch():
            @pl.when(b >= 1)
            def _():
                drain_stores(nslot)
            issue_gathers(b + 1, nslot)

        wait_gathers(b, slot)
        add_pos(slot)
        issue_stores(b, slot)
        return carry

    lax.fori_loop(0, nb, batch_body, 0)
    drain_stores(lax.rem(nb - 2, 2))
    drain_stores(lax.rem(nb - 1, 2))


def _tail_body(x_ref, tail_ref, pos_ref, o_ref):
    o_ref[...] = tail_ref[...] + pos_ref[...][None, :, :]


def _tail_fix(out_sc, tail_tok, position_table):
    return pl.pallas_call(
        _tail_body,
        out_shape=jax.ShapeDtypeStruct((BATCH, SEQ, HIDDEN), jnp.float32),
        grid=(BATCH // BATCH_BLOCK,),
        in_specs=[
            pl.BlockSpec((1, TAIL, HIDDEN), lambda b: (b, 9, 0)),
            pl.BlockSpec((BATCH_BLOCK, TAIL, HIDDEN), lambda b: (b, 0, 0)),
            pl.BlockSpec((TAIL, HIDDEN), lambda b: (9, 0)),
        ],
        out_specs=pl.BlockSpec((BATCH_BLOCK, TAIL, HIDDEN), lambda b: (b, 9, 0)),
        input_output_aliases={0: 0},
    )(out_sc, tail_tok, position_table)


@jax.jit
def _embed(ids, token_table, position_table):
    tail_ids = jnp.pad(ids[:, FULL_ROWS:], ((0, 0), (0, TAIL - (SEQ - FULL_ROWS))))
    pos_pad = jnp.pad(position_table, ((0, TAIL * 10 - SEQ), (0, 0)))
    mesh = plsc.VectorSubcoreMesh(
        core_axis_name="c", subcore_axis_name="s",
        num_cores=NUM_CORES, num_subcores=NUM_SUBCORES,
    )
    f = pl.kernel(
        _embed_body,
        out_type=(
            jax.ShapeDtypeStruct((BATCH, SEQ, HIDDEN), jnp.float32),
            jax.ShapeDtypeStruct((BATCH, TAIL, HIDDEN), jnp.float32),
        ),
        mesh=mesh,
        scratch_types=[
            pltpu.VMEM((CHUNK_A, HIDDEN), jnp.float32),
            pltpu.VMEM((BATCHES_PER_PAIR, SEQ), jnp.int32),
            pltpu.VMEM((BATCHES_PER_PAIR, TAIL), jnp.int32),
            pltpu.VMEM((2, CHUNK_A, HIDDEN), jnp.float32),
            pltpu.VMEM((2, TAIL, HIDDEN), jnp.float32),
            pltpu.SemaphoreType.DMA,
            pltpu.SemaphoreType.DMA,
            pltpu.SemaphoreType.DMA,
            pltpu.SemaphoreType.DMA,
        ],
    )
    out_sc, tail_tok = f(ids, tail_ids, token_table, pos_pad)
    return _tail_fix(out_sc, tail_tok, position_table)


def kernel(input_ids, token_table, position_table):
    ids = input_ids.astype(jnp.int32)
    return _embed(ids, token_table, position_table)


# speedup vs baseline: 1.0055x; 1.0039x over previous
import jax
import jax.numpy as jnp
from jax import lax
from jax.experimental import pallas as pl
from jax.experimental.pallas import tpu as pltpu
from jax.experimental.pallas import tpu_sc as plsc

VOCAB = 49408
HIDDEN = 768
SEQ = 77
BATCH = 1024
LANES = 16
NVEC = HIDDEN // LANES

NUM_CORES = 2
NUM_SUBCORES = 16
NUM_WORKERS = NUM_CORES * NUM_SUBCORES
BATCHES_PER_WORKER = BATCH // NUM_WORKERS

FULL_ROWS = 72
CHUNK_A = 40
CHUNK_B = 32
TAIL = 8
BATCH_BLOCK = 8


def _embed_body(ids_hbm, tids_hbm, tok_hbm, pos_hbm, out_hbm, tail_hbm,
                pos_v, idx_v, tidx_v, buf_a, buf_b, buf_t,
                gsem_a, gsem_b, gsem_t, ssem_a, ssem_b, ssem_t):
    cid = lax.axis_index("c")
    sid = lax.axis_index("s")
    wid = sid * NUM_CORES + cid
    base_b = wid * BATCHES_PER_WORKER

    pltpu.sync_copy(pos_hbm.at[pl.ds(0, FULL_ROWS)], pos_v)

    def add_pos(buf, nrows, pos_off):
        def row_body(r, carry):
            for c in range(NVEC):
                sl = pl.ds(c * LANES, LANES)
                buf[r, sl] = buf[r, sl] + pos_v[pos_off + r, sl]
            return carry
        lax.fori_loop(0, nrows, row_body, 0)

    def batch_body(i, carry):
        gb = base_b + i
        pltpu.sync_copy(ids_hbm.at[gb], idx_v)
        pltpu.sync_copy(tids_hbm.at[gb], tidx_v)
        ga = pltpu.async_copy(tok_hbm.at[idx_v.at[pl.ds(0, CHUNK_A)]], buf_a, gsem_a)
        gb_ = pltpu.async_copy(tok_hbm.at[idx_v.at[pl.ds(CHUNK_A, CHUNK_B)]], buf_b, gsem_b)
        gt = pltpu.async_copy(tok_hbm.at[tidx_v], buf_t, gsem_t)

        ga.wait()
        add_pos(buf_a, CHUNK_A, 0)
        sa = pltpu.async_copy(buf_a, out_hbm.at[gb, pl.ds(0, CHUNK_A)], ssem_a)
        gb_.wait()
        add_pos(buf_b, CHUNK_B, CHUNK_A)
        sb = pltpu.async_copy(buf_b, out_hbm.at[gb, pl.ds(CHUNK_A, CHUNK_B)], ssem_b)
        gt.wait()
        st = pltpu.async_copy(buf_t, tail_hbm.at[gb], ssem_t)
        sa.wait()
        sb.wait()
        st.wait()
        return carry

    lax.fori_loop(0, BATCHES_PER_WORKER, batch_body, 0)


def _tail_body(x_ref, tail_ref, pos_ref, o_ref):
    o_ref[...] = tail_ref[...] + pos_ref[...][None, :, :]


def _tail_fix(out_sc, tail_tok, position_table):
    return pl.pallas_call(
        _tail_body,
        out_shape=jax.ShapeDtypeStruct((BATCH, SEQ, HIDDEN), jnp.float32),
        grid=(BATCH // BATCH_BLOCK,),
        in_specs=[
            pl.BlockSpec((1, TAIL, HIDDEN), lambda b: (b, 9, 0)),
            pl.BlockSpec((BATCH_BLOCK, TAIL, HIDDEN), lambda b: (b, 0, 0)),
            pl.BlockSpec((TAIL, HIDDEN), lambda b: (9, 0)),
        ],
        out_specs=pl.BlockSpec((BATCH_BLOCK, TAIL, HIDDEN), lambda b: (b, 9, 0)),
        input_output_aliases={0: 0},
    )(out_sc, tail_tok, position_table)


@jax.jit
def _embed(ids, token_table, position_table):
    tail_ids = jnp.pad(ids[:, FULL_ROWS:], ((0, 0), (0, TAIL - (SEQ - FULL_ROWS))))
    mesh = plsc.VectorSubcoreMesh(
        core_axis_name="c", subcore_axis_name="s",
        num_cores=NUM_CORES, num_subcores=NUM_SUBCORES,
    )
    f = pl.kernel(
        _embed_body,
        out_type=(
            jax.ShapeDtypeStruct((BATCH, SEQ, HIDDEN), jnp.float32),
            jax.ShapeDtypeStruct((BATCH, TAIL, HIDDEN), jnp.float32),
        ),
        mesh=mesh,
        scratch_types=[
            pltpu.VMEM((FULL_ROWS, HIDDEN), jnp.float32),
            pltpu.VMEM((SEQ,), jnp.int32),
            pltpu.VMEM((TAIL,), jnp.int32),
            pltpu.VMEM((CHUNK_A, HIDDEN), jnp.float32),
            pltpu.VMEM((CHUNK_B, HIDDEN), jnp.float32),
            pltpu.VMEM((TAIL, HIDDEN), jnp.float32),
            pltpu.SemaphoreType.DMA,
            pltpu.SemaphoreType.DMA,
            pltpu.SemaphoreType.DMA,
            pltpu.SemaphoreType.DMA,
            pltpu.SemaphoreType.DMA,
            pltpu.SemaphoreType.DMA,
        ],
    )
    out_sc, tail_tok = f(ids, tail_ids, token_table, position_table)
    return _tail_fix(out_sc, tail_tok, position_table)


def kernel(input_ids, token_table, position_table):
    ids = input_ids.astype(jnp.int32)
    return _embed(ids, token_table, position_table)
